# baseline (speedup 1.0000x reference)
import functools

import jax
import jax.numpy as jnp
from jax.experimental import pallas as pl
from jax.experimental.pallas import tpu as pltpu

B, S, D, E, K = 2, 4096, 2048, 64, 2
TOK_BLK = 2048
DH = D // 2


def _router_kernel(xa_ref, xb_ref, wa_ref, wb_ref, b_ref, out_ref):
    h = jnp.dot(xa_ref[...], wa_ref[...], preferred_element_type=jnp.float32)
    h = h + jnp.dot(xb_ref[...], wb_ref[...], preferred_element_type=jnp.float32)
    h = h + b_ref[...]
    neg_inf = jnp.float32(-jnp.inf)
    iota = jax.lax.broadcasted_iota(jnp.int32, h.shape, 1)
    m1 = jnp.max(h, axis=1, keepdims=True)
    i1 = jnp.min(jnp.where(h == m1, iota, E), axis=1, keepdims=True)
    sel1 = iota == i1
    h2 = jnp.where(sel1, neg_inf, h)
    m2 = jnp.max(h2, axis=1, keepdims=True)
    i2 = jnp.min(jnp.where(h2 == m2, iota, E), axis=1, keepdims=True)
    sel2 = iota == i2
    e2 = jnp.exp(m2 - m1)
    z = 1.0 + e2
    out_ref[...] = jnp.where(sel1, 1.0 / z, jnp.where(sel2, e2 / z, 0.0))


@functools.partial(jax.jit, static_argnames=())
def kernel(x, W, b):
    xt = x.reshape(B * S, D)
    wt = W.T
    b2 = b.reshape(1, E)
    grid = (B * S) // TOK_BLK
    out = pl.pallas_call(
        _router_kernel,
        grid=(grid,),
        in_specs=[
            pl.BlockSpec((TOK_BLK, DH), lambda i: (i, 0)),
            pl.BlockSpec((TOK_BLK, DH), lambda i: (i, 1)),
            pl.BlockSpec((DH, E), lambda i: (0, 0)),
            pl.BlockSpec((DH, E), lambda i: (1, 0)),
            pl.BlockSpec((1, E), lambda i: (0, 0)),
        ],
        out_specs=pl.BlockSpec((TOK_BLK, E), lambda i: (i, 0)),
        out_shape=jax.ShapeDtypeStruct((B * S, E), jnp.float32),
        compiler_params=pltpu.CompilerParams(
            dimension_semantics=("parallel",),
        ),
    )(xt, xt, wt, wt, b2)
    return out.reshape(B, S, E)


# speedup vs baseline: 1.0174x; 1.0174x over previous
import functools

import jax
import jax.numpy as jnp
from jax.experimental import pallas as pl
from jax.experimental.pallas import tpu as pltpu

B, S, D, E, K = 2, 4096, 2048, 64, 2
TOK_BLK = 2048


def _router_kernel(x_ref, wt_ref, b_ref, out_ref):
    h = jnp.dot(x_ref[...], wt_ref[...], preferred_element_type=jnp.float32)
    h = h + b_ref[...]
    neg_inf = jnp.float32(-jnp.inf)
    lane = jax.lax.broadcasted_iota(jnp.int32, h.shape, 1).astype(jnp.float32)
    m1 = jnp.max(h, axis=1, keepdims=True)
    t1 = jnp.where(h == m1, lane, jnp.float32(E))
    i1 = jnp.min(t1, axis=1, keepdims=True)
    sel1 = t1 == i1
    h2 = jnp.where(sel1, neg_inf, h)
    m2 = jnp.max(h2, axis=1, keepdims=True)
    t2 = jnp.where(h2 == m2, lane, jnp.float32(E))
    i2 = jnp.min(t2, axis=1, keepdims=True)
    sel2 = t2 == i2
    e2 = jnp.exp(m2 - m1)
    z = 1.0 + e2
    out_ref[...] = jnp.where(sel1, 1.0 / z, jnp.where(sel2, e2 / z, 0.0))


@functools.partial(jax.jit, static_argnames=())
def kernel(x, W, b):
    xt = x.reshape(B * S, D)
    wt = W.T
    b2 = b.reshape(1, E)
    grid = (B * S) // TOK_BLK
    out = pl.pallas_call(
        _router_kernel,
        grid=(grid,),
        in_specs=[
            pl.BlockSpec((TOK_BLK, D), lambda i: (i, 0)),
            pl.BlockSpec((D, E), lambda i: (0, 0)),
            pl.BlockSpec((1, E), lambda i: (0, 0)),
        ],
        out_specs=pl.BlockSpec((TOK_BLK, E), lambda i: (i, 0)),
        out_shape=jax.ShapeDtypeStruct((B * S, E), jnp.float32),
        compiler_params=pltpu.CompilerParams(
            dimension_semantics=("parallel",),
        ),
    )(xt, wt, b2)
    return out.reshape(B, S, E)


# speedup vs baseline: 1.0304x; 1.0128x over previous
import functools

import jax
import jax.numpy as jnp
from jax.experimental import pallas as pl
from jax.experimental.pallas import tpu as pltpu

B, S, D, E, K = 2, 4096, 2048, 64, 2
TOK_BLK = 1024


def _router_kernel(x_ref, wt_ref, b_ref, out_ref):
    h = jnp.dot(x_ref[...], wt_ref[...], preferred_element_type=jnp.float32)
    h = h + b_ref[...]
    neg_inf = jnp.float32(-jnp.inf)
    lane = jax.lax.broadcasted_iota(jnp.int32, h.shape, 1).astype(jnp.float32)
    m1 = jnp.max(h, axis=1, keepdims=True)
    t1 = jnp.where(h == m1, lane, jnp.float32(E))
    i1 = jnp.min(t1, axis=1, keepdims=True)
    sel1 = t1 == i1
    h2 = jnp.where(sel1, neg_inf, h)
    m2 = jnp.max(h2, axis=1, keepdims=True)
    t2 = jnp.where(h2 == m2, lane, jnp.float32(E))
    i2 = jnp.min(t2, axis=1, keepdims=True)
    sel2 = t2 == i2
    e2 = jnp.exp(m2 - m1)
    z = 1.0 + e2
    out_ref[...] = jnp.where(sel1, 1.0 / z, jnp.where(sel2, e2 / z, 0.0))


@functools.partial(jax.jit, static_argnames=())
def kernel(x, W, b):
    xt = x.reshape(B * S, D)
    wt = W.T
    b2 = b.reshape(1, E)
    grid = (B * S) // TOK_BLK
    out = pl.pallas_call(
        _router_kernel,
        grid=(grid,),
        in_specs=[
            pl.BlockSpec((TOK_BLK, D), lambda i: (i, 0)),
            pl.BlockSpec((D, E), lambda i: (0, 0)),
            pl.BlockSpec((1, E), lambda i: (0, 0)),
        ],
        out_specs=pl.BlockSpec((TOK_BLK, E), lambda i: (i, 0)),
        out_shape=jax.ShapeDtypeStruct((B * S, E), jnp.float32),
        compiler_params=pltpu.CompilerParams(
            dimension_semantics=("parallel",),
        ),
    )(xt, wt, b2)
    return out.reshape(B, S, E)
